# baseline (speedup 1.0000x reference)
import functools

import jax
import jax.numpy as jnp
from jax import lax
from jax.experimental import pallas as pl
from jax.experimental.pallas import tpu as pltpu
from jax.experimental.pallas import tpu_sc as plsc

_BS, _SEQ, _D = 4, 8192, 768
_MAX_LEN = 32768
_B = _BS * _SEQ

_NC, _NS = 2, 16
_NW = _NC * _NS
_B_PER_W = _B // _NW
_NBUF = 10
_CHUNK = 16
_N_CHUNKS = _B_PER_W // _CHUNK

_mesh = plsc.VectorSubcoreMesh(core_axis_name="c", subcore_axis_name="s")


@functools.partial(
    pl.kernel,
    mesh=_mesh,
    out_type=jax.ShapeDtypeStruct((_B, _D), jnp.float32),
    scratch_types=[
        pltpu.VMEM((_B_PER_W,), jnp.int32),
    ]
    + [pltpu.VMEM((_CHUNK, _D), jnp.float32) for _ in range(_NBUF)]
    + [pltpu.SemaphoreType.DMA for _ in range(2 * _NBUF)],
)
def _sc_gather(idx_hbm, table_hbm, out_hbm, idx_v, *scratch):
    bufs = scratch[:_NBUF]
    gsem = scratch[_NBUF : 2 * _NBUF]
    psem = scratch[2 * _NBUF :]
    wid = lax.axis_index("s") * _NC + lax.axis_index("c")
    base = wid * _B_PER_W
    b = wid // (_SEQ // _B_PER_W)
    off = (wid % (_SEQ // _B_PER_W)) * _B_PER_W
    pltpu.sync_copy(idx_hbm.at[b, pl.ds(off, _B_PER_W)], idx_v)

    def gather(c):
        s = c % _NBUF
        return pltpu.async_copy(
            table_hbm.at[idx_v.at[pl.ds(c * _CHUNK, _CHUNK)]], bufs[s], gsem[s]
        )

    def put(c):
        s = c % _NBUF
        return pltpu.async_copy(
            bufs[s], out_hbm.at[pl.ds(base + c * _CHUNK, _CHUNK)], psem[s]
        )

    gathers = [None] * _N_CHUNKS
    puts = [None] * _N_CHUNKS
    for g in range(_NBUF - 1):
        gathers[g] = gather(g)
    for g in range(_N_CHUNKS):
        gathers[g].wait()
        puts[g] = put(g)
        if g + _NBUF - 1 < _N_CHUNKS:
            if g >= 1:
                puts[g - 1].wait()
            gathers[g + _NBUF - 1] = gather(g + _NBUF - 1)
    for g in range(max(0, _N_CHUNKS - _NBUF), _N_CHUNKS):
        puts[g].wait()


_DMA_WINDOW = 16


@functools.partial(
    pl.kernel,
    mesh=_mesh,
    out_type=jax.ShapeDtypeStruct((_B, _D), jnp.float32),
    scratch_types=[
        pltpu.SMEM((_B_PER_W,), jnp.int32),
        pltpu.VMEM((_B_PER_W,), jnp.int32),
        pltpu.VMEM_SHARED((_NS, _B_PER_W), jnp.int32),
        pltpu.SemaphoreType.DMA,
    ],
)
def _sc_gather_dma(idx_hbm, table_hbm, out_hbm, idx_s, idx_v, idx_sp, sem):
    cid = lax.axis_index("c")
    sid = lax.axis_index("s")
    wid = sid * _NC + cid
    base = wid * _B_PER_W
    pltpu.sync_copy(idx_hbm.at[pl.ds(base, _B_PER_W)], idx_v)
    pltpu.sync_copy(idx_v, idx_sp.at[sid])
    pltpu.sync_copy(idx_sp.at[sid], idx_s)

    def row_wait():
        pltpu.make_async_copy(
            table_hbm.at[pl.ds(0, 1)], out_hbm.at[pl.ds(base, 1)], sem
        ).wait()

    def body(i, carry):
        idx = idx_s[i]
        pltpu.async_copy(
            table_hbm.at[pl.ds(idx, 1)], out_hbm.at[pl.ds(base + i, 1)], sem
        )

        @pl.when(i >= _DMA_WINDOW)
        def _():
            row_wait()

        return carry

    lax.fori_loop(0, _B_PER_W, body, 0)
    for _ in range(_DMA_WINDOW):
        row_wait()


def kernel(inputs, input_positions, pos_embedding):
    del inputs
    table = pos_embedding.reshape(_MAX_LEN, _D)
    out = _sc_gather(input_positions.astype(jnp.int32), table)
    return out.reshape(1, _BS, _SEQ, _D)


# speedup vs baseline: 1.0094x; 1.0094x over previous
import functools

import jax
import jax.numpy as jnp
from jax import lax
from jax.experimental import pallas as pl
from jax.experimental.pallas import tpu as pltpu
from jax.experimental.pallas import tpu_sc as plsc

_BS, _SEQ, _D = 4, 8192, 768
_MAX_LEN = 32768
_B = _BS * _SEQ

_NC, _NS = 2, 16
_NW = _NC * _NS
_B_PER_W = _B // _NW
_NBUF = 5
_CHUNK = 32
_N_CHUNKS = _B_PER_W // _CHUNK

_mesh = plsc.VectorSubcoreMesh(core_axis_name="c", subcore_axis_name="s")


@functools.partial(
    pl.kernel,
    mesh=_mesh,
    out_type=jax.ShapeDtypeStruct((_B, _D), jnp.float32),
    scratch_types=[
        pltpu.VMEM((_B_PER_W,), jnp.int32),
    ]
    + [pltpu.VMEM((_CHUNK, _D), jnp.float32) for _ in range(_NBUF)]
    + [pltpu.SemaphoreType.DMA for _ in range(2 * _NBUF)],
)
def _sc_gather(idx_hbm, table_hbm, out_hbm, idx_v, *scratch):
    bufs = scratch[:_NBUF]
    gsem = scratch[_NBUF : 2 * _NBUF]
    psem = scratch[2 * _NBUF :]
    wid = lax.axis_index("s") * _NC + lax.axis_index("c")
    base = wid * _B_PER_W
    b = wid // (_SEQ // _B_PER_W)
    off = (wid % (_SEQ // _B_PER_W)) * _B_PER_W
    pltpu.sync_copy(idx_hbm.at[b, pl.ds(off, _B_PER_W)], idx_v)

    def gather(c):
        s = c % _NBUF
        return pltpu.async_copy(
            table_hbm.at[idx_v.at[pl.ds(c * _CHUNK, _CHUNK)]], bufs[s], gsem[s]
        )

    def put(c):
        s = c % _NBUF
        return pltpu.async_copy(
            bufs[s], out_hbm.at[pl.ds(base + c * _CHUNK, _CHUNK)], psem[s]
        )

    gathers = [None] * _N_CHUNKS
    puts = [None] * _N_CHUNKS
    for g in range(_NBUF - 1):
        gathers[g] = gather(g)
    for g in range(_N_CHUNKS):
        gathers[g].wait()
        puts[g] = put(g)
        if g + _NBUF - 1 < _N_CHUNKS:
            if g >= 1:
                puts[g - 1].wait()
            gathers[g + _NBUF - 1] = gather(g + _NBUF - 1)
    for g in range(max(0, _N_CHUNKS - _NBUF), _N_CHUNKS):
        puts[g].wait()


def kernel(inputs, input_positions, pos_embedding):
    del inputs
    table = pos_embedding.reshape(_MAX_LEN, _D)
    out = _sc_gather(input_positions.astype(jnp.int32), table)
    return out.reshape(1, _BS, _SEQ, _D)
